# Initial kernel scaffold; baseline (speedup 1.0000x reference)
import jax
import jax.numpy as jnp
from jax.experimental import pallas as pl


def kernel(hidden_state, W1, b1, W2, b2, W3, b3):
    raise NotImplementedError("write your pallas kernel here")


# speedup vs baseline: 2.4190x; 2.4190x over previous
import jax
import jax.numpy as jnp
from jax.experimental import pallas as pl
from jax.experimental.pallas import tpu as pltpu


def _body(h_ref, w1a_ref, w1b_ref, b1_ref, w2_ref, b2_ref, w3_ref, b3_ref,
          o_ref):
    h = h_ref[0]
    T = h.shape[0]
    ri = jax.lax.broadcasted_iota(jnp.int32, (T, T), 0)
    ci = jax.lax.broadcasted_iota(jnp.int32, (T, T), 1)
    rf = jnp.maximum(ri, 1).astype(jnp.float32)
    coeff = jnp.where(ci < ri, 1.0 / rf, 0.0)
    pm = jnp.dot(coeff, h, preferred_element_type=jnp.float32)
    z1 = (jnp.dot(pm, w1a_ref[...], preferred_element_type=jnp.float32)
          + jnp.dot(h, w1b_ref[...], preferred_element_type=jnp.float32)
          + b1_ref[...])
    h1 = jnp.maximum(z1, 0.0)
    z2 = jnp.dot(h1, w2_ref[...], preferred_element_type=jnp.float32) + b2_ref[...]
    h2 = jnp.maximum(z2, 0.0)
    row = jnp.sum(h2 * w3_ref[...], axis=1) + b3_ref[0, 0]
    o_ref[...] = row.reshape(1, 1, T)


def kernel(hidden_state, W1, b1, W2, b2, W3, b3):
    B, T, D = hidden_state.shape
    H2 = W1.shape[1]
    H = W2.shape[1]
    W1a = W1[:D]
    W1b = W1[D:]
    out = pl.pallas_call(
        _body,
        grid=(B,),
        in_specs=[
            pl.BlockSpec((1, T, D), lambda b: (b, 0, 0)),
            pl.BlockSpec((D, H2), lambda b: (0, 0)),
            pl.BlockSpec((D, H2), lambda b: (0, 0)),
            pl.BlockSpec((1, H2), lambda b: (0, 0)),
            pl.BlockSpec((H2, H), lambda b: (0, 0)),
            pl.BlockSpec((1, H), lambda b: (0, 0)),
            pl.BlockSpec((1, H), lambda b: (0, 0)),
            pl.BlockSpec((1, 1), lambda b: (0, 0)),
        ],
        out_specs=pl.BlockSpec((1, 1, T), lambda b: (b, 0, 0)),
        out_shape=jax.ShapeDtypeStruct((B, 1, T), jnp.float32),
        compiler_params=pltpu.CompilerParams(
            dimension_semantics=("parallel",),
            vmem_limit_bytes=64 * 1024 * 1024,
        ),
        name="ffd_fused",
    )(hidden_state, W1a, W1b, b1.reshape(1, H2), W2, b2.reshape(1, H),
      W3.reshape(1, H), b3.reshape(1, 1))
    return out.reshape(B, T, 1)


# speedup vs baseline: 2.4730x; 1.0223x over previous
import jax
import jax.numpy as jnp
from jax.experimental import pallas as pl
from jax.experimental.pallas import tpu as pltpu


def _body(h_ref, w1a_ref, w1b_ref, b1_ref, w2_ref, b2_ref, w3_ref, b3_ref,
          o_ref):
    h = h_ref[0]
    T = h.shape[0]
    ri = jax.lax.broadcasted_iota(jnp.int32, (T, T), 0)
    ci = jax.lax.broadcasted_iota(jnp.int32, (T, T), 1)
    rf = jnp.maximum(ri, 1).astype(jnp.float32)
    coeff = jnp.where(ci < ri, 1.0 / rf, 0.0)
    pm = jnp.dot(coeff, h, preferred_element_type=jnp.float32)
    z1 = (jnp.dot(pm, w1a_ref[...], preferred_element_type=jnp.float32)
          + jnp.dot(h, w1b_ref[...], preferred_element_type=jnp.float32)
          + b1_ref[...])
    h1 = jnp.maximum(z1, 0.0)
    z2 = jnp.dot(h1, w2_ref[...], preferred_element_type=jnp.float32) + b2_ref[...]
    h2 = jnp.maximum(z2, 0.0)
    row = jnp.sum(h2 * w3_ref[...], axis=1) + b3_ref[0, 0]
    o_ref[...] = row.reshape(1, 1, T)


def kernel(hidden_state, W1, b1, W2, b2, W3, b3):
    B, T, D = hidden_state.shape
    H2 = W1.shape[1]
    H = W2.shape[1]
    out = pl.pallas_call(
        _body,
        grid=(B,),
        in_specs=[
            pl.BlockSpec((1, T, D), lambda b: (b, 0, 0)),
            pl.BlockSpec((D, H2), lambda b: (0, 0)),
            pl.BlockSpec((D, H2), lambda b: (1, 0)),
            pl.BlockSpec((1, H2), lambda b: (0, 0)),
            pl.BlockSpec((H2, H), lambda b: (0, 0)),
            pl.BlockSpec((1, H), lambda b: (0, 0)),
            pl.BlockSpec((1, H), lambda b: (0, 0)),
            pl.BlockSpec((1, 1), lambda b: (0, 0)),
        ],
        out_specs=pl.BlockSpec((1, 1, T), lambda b: (b, 0, 0)),
        out_shape=jax.ShapeDtypeStruct((B, 1, T), jnp.float32),
        compiler_params=pltpu.CompilerParams(
            dimension_semantics=("arbitrary",),
            vmem_limit_bytes=64 * 1024 * 1024,
        ),
        name="ffd_fused",
    )(hidden_state, W1, W1, b1.reshape(1, H2), W2, b2.reshape(1, H),
      W3.reshape(1, H), b3.reshape(1, 1))
    return out.reshape(B, T, 1)
